# Initial kernel scaffold; baseline (speedup 1.0000x reference)
import jax
import jax.numpy as jnp
from jax.experimental import pallas as pl


def kernel(inputs, edge_index, W1, b1, W2, b2):
    raise NotImplementedError("write your pallas kernel here")


# speedup vs baseline: 12.5615x; 12.5615x over previous
import functools

import jax
import jax.numpy as jnp
from jax import lax
from jax.experimental import pallas as pl
from jax.experimental.pallas import tpu as pltpu
from jax.experimental.pallas import tpu_sc as plsc

N = 10000
EDGES = 320000
F_IN = 128
F_H = 64

N_PAD = 10240
N_PER_TILE = N_PAD // 16
CHUNK = 128
N_TILES = 32
CPT = 80
E_PAD = N_TILES * CPT * CHUNK
ROW_BLK = 1280


def _mesh():
    return plsc.VectorSubcoreMesh(core_axis_name="c", subcore_axis_name="s")


def _sc_degrees(src2d, dst2d):
    def body(src_h, dst_h, dout_h, din_h, srcv, dstv, ones_v, zv, acc_o, acc_i):
        c = lax.axis_index("c")
        s = lax.axis_index("s")
        tid = s * 2 + c

        def set_ones(i, _):
            ones_v[pl.ds(i * 16, 16)] = jnp.ones((16,), jnp.float32)
            return 0

        lax.fori_loop(0, CHUNK // 16, set_ones, 0)

        def set_zero(i, _):
            zv[pl.ds(i * 16, 16)] = jnp.zeros((16,), jnp.float32)
            return 0

        lax.fori_loop(0, N_PER_TILE // 16, set_zero, 0)

        sl = pl.ds(s * N_PER_TILE, N_PER_TILE)
        pltpu.sync_copy(zv, acc_o.at[sl])
        pltpu.sync_copy(zv, acc_i.at[sl])
        plsc.subcore_barrier()

        pltpu.sync_copy(src_h.at[pl.ds(tid * CPT, CPT)], srcv)
        pltpu.sync_copy(dst_h.at[pl.ds(tid * CPT, CPT)], dstv)

        def step(j, _):
            pltpu.sync_copy(ones_v, acc_o.at[srcv.at[j]], add=True)
            pltpu.sync_copy(ones_v, acc_i.at[dstv.at[j]], add=True)
            return 0

        lax.fori_loop(0, CPT, step, 0)
        plsc.subcore_barrier()

        pltpu.sync_copy(acc_o.at[sl], dout_h.at[c, sl])
        pltpu.sync_copy(acc_i.at[sl], din_h.at[c, sl])

    return pl.kernel(
        body,
        out_type=[
            jax.ShapeDtypeStruct((2, N_PAD), jnp.float32),
            jax.ShapeDtypeStruct((2, N_PAD), jnp.float32),
        ],
        mesh=_mesh(),
        scratch_types=[
            pltpu.VMEM((CPT, CHUNK), jnp.int32),
            pltpu.VMEM((CPT, CHUNK), jnp.int32),
            pltpu.VMEM((CHUNK,), jnp.float32),
            pltpu.VMEM((N_PER_TILE,), jnp.float32),
            pltpu.VMEM_SHARED((N_PAD,), jnp.float32),
            pltpu.VMEM_SHARED((N_PAD,), jnp.float32),
        ],
    )(src2d, dst2d)


def _sc_agg(y, src2d, dst2d):
    def body(y_h, src_h, dst_h, out_h, srcv, dstv, rows0, rows1, zbuf, acc,
             sem0, sem1):
        c = lax.axis_index("c")
        s = lax.axis_index("s")
        tid = s * 2 + c

        def zb(i, _):
            zbuf[i // 4, pl.ds((i % 4) * 16, 16)] = jnp.zeros((16,), jnp.float32)
            return 0

        lax.fori_loop(0, CHUNK * 4, zb, 0)

        def zc(i, _):
            pltpu.sync_copy(zbuf, acc.at[pl.ds(s * N_PER_TILE + i * CHUNK, CHUNK)])
            return 0

        lax.fori_loop(0, N_PER_TILE // CHUNK, zc, 0)
        plsc.subcore_barrier()

        pltpu.sync_copy(src_h.at[pl.ds(tid * CPT, CPT)], srcv)
        pltpu.sync_copy(dst_h.at[pl.ds(tid * CPT, CPT)], dstv)

        pltpu.async_copy(y_h.at[srcv.at[0]], rows0, sem0)

        def step(g, _):
            j0 = g * 2
            pltpu.async_copy(y_h.at[srcv.at[j0 + 1]], rows1, sem1)
            pltpu.make_async_copy(y_h.at[srcv.at[j0]], rows0, sem0).wait()
            pltpu.sync_copy(rows0, acc.at[dstv.at[j0]], add=True)

            @pl.when(g < CPT // 2 - 1)
            def _issue_next():
                pltpu.async_copy(y_h.at[srcv.at[j0 + 2]], rows0, sem0)

            pltpu.make_async_copy(y_h.at[srcv.at[j0 + 1]], rows1, sem1).wait()
            pltpu.sync_copy(rows1, acc.at[dstv.at[j0 + 1]], add=True)
            return 0

        lax.fori_loop(0, CPT // 2, step, 0)
        plsc.subcore_barrier()

        sl = pl.ds(s * N_PER_TILE, N_PER_TILE)
        pltpu.sync_copy(acc.at[sl], out_h.at[c, sl])

    return pl.kernel(
        body,
        out_type=jax.ShapeDtypeStruct((2, N_PAD, F_H), jnp.float32),
        mesh=_mesh(),
        scratch_types=[
            pltpu.VMEM((CPT, CHUNK), jnp.int32),
            pltpu.VMEM((CPT, CHUNK), jnp.int32),
            pltpu.VMEM((CHUNK, F_H), jnp.float32),
            pltpu.VMEM((CHUNK, F_H), jnp.float32),
            pltpu.VMEM((CHUNK, F_H), jnp.float32),
            pltpu.VMEM_SHARED((N_PAD, F_H), jnp.float32),
            pltpu.SemaphoreType.DMA,
            pltpu.SemaphoreType.DMA,
        ],
        compiler_params=pltpu.CompilerParams(use_tc_tiling_on_sc=False),
    )(y, src2d, dst2d)


def _tc_mm1(x, do0, do1, W1):
    def body(x_ref, d0, d1, w_ref, o_ref):
        ns = lax.rsqrt(jnp.maximum(d0[...] + d1[...], 1.0))
        o_ref[...] = jnp.dot(x_ref[...] * ns, w_ref[...],
                             preferred_element_type=jnp.float32)

    return pl.pallas_call(
        body,
        grid=(N_PAD // ROW_BLK,),
        in_specs=[
            pl.BlockSpec((ROW_BLK, F_IN), lambda i: (i, 0)),
            pl.BlockSpec((ROW_BLK, 1), lambda i: (i, 0)),
            pl.BlockSpec((ROW_BLK, 1), lambda i: (i, 0)),
            pl.BlockSpec((F_IN, F_H), lambda i: (0, 0)),
        ],
        out_specs=pl.BlockSpec((ROW_BLK, F_H), lambda i: (i, 0)),
        out_shape=jax.ShapeDtypeStruct((N_PAD, F_H), jnp.float32),
    )(x, do0, do1, W1)


def _tc_mid(a0, a1, di0, di1, do0, do1, b1, W2):
    def body(a0r, a1r, i0, i1, o0, o1, br, w_ref, o_ref):
        nd = lax.rsqrt(jnp.maximum(i0[...] + i1[...], 1.0))
        h = jnp.maximum((a0r[...] + a1r[...]) * nd + br[...], 0.0)
        ns = lax.rsqrt(jnp.maximum(o0[...] + o1[...], 1.0))
        o_ref[...] = jnp.dot(h * ns, w_ref[...],
                             preferred_element_type=jnp.float32)

    rb = lambda i: (i, 0)
    return pl.pallas_call(
        body,
        grid=(N_PAD // ROW_BLK,),
        in_specs=[
            pl.BlockSpec((ROW_BLK, F_H), rb),
            pl.BlockSpec((ROW_BLK, F_H), rb),
            pl.BlockSpec((ROW_BLK, 1), rb),
            pl.BlockSpec((ROW_BLK, 1), rb),
            pl.BlockSpec((ROW_BLK, 1), rb),
            pl.BlockSpec((ROW_BLK, 1), rb),
            pl.BlockSpec((1, F_H), lambda i: (0, 0)),
            pl.BlockSpec((F_H, F_H), lambda i: (0, 0)),
        ],
        out_specs=pl.BlockSpec((ROW_BLK, F_H), rb),
        out_shape=jax.ShapeDtypeStruct((N_PAD, F_H), jnp.float32),
    )(a0, a1, di0, di1, do0, do1, b1, W2)


def _tc_final(a0, a1, di0, di1, b2):
    def body(a0r, a1r, i0, i1, br, o_ref):
        nd = lax.rsqrt(jnp.maximum(i0[...] + i1[...], 1.0))
        o_ref[...] = jnp.maximum((a0r[...] + a1r[...]) * nd + br[...], 0.0)

    rb = lambda i: (i, 0)
    return pl.pallas_call(
        body,
        grid=(N_PAD // ROW_BLK,),
        in_specs=[
            pl.BlockSpec((ROW_BLK, F_H), rb),
            pl.BlockSpec((ROW_BLK, F_H), rb),
            pl.BlockSpec((ROW_BLK, 1), rb),
            pl.BlockSpec((ROW_BLK, 1), rb),
            pl.BlockSpec((1, F_H), lambda i: (0, 0)),
        ],
        out_specs=pl.BlockSpec((ROW_BLK, F_H), rb),
        out_shape=jax.ShapeDtypeStruct((N_PAD, F_H), jnp.float32),
    )(a0, a1, di0, di1, b2)


def kernel(inputs, edge_index, W1, b1, W2, b2):
    src = edge_index[0]
    dst = edge_index[1]
    pad = E_PAD - EDGES
    padidx = (N + (jnp.arange(pad, dtype=jnp.int32) % (N_PAD - N))).astype(jnp.int32)
    src2d = jnp.concatenate([src, padidx]).reshape(E_PAD // CHUNK, CHUNK)
    dst2d = jnp.concatenate([dst, padidx]).reshape(E_PAD // CHUNK, CHUNK)
    x_p = jnp.concatenate(
        [inputs, jnp.zeros((N_PAD - N, F_IN), jnp.float32)], axis=0)

    degp_out, degp_in = _sc_degrees(src2d, dst2d)
    do0 = degp_out[0].reshape(N_PAD, 1)
    do1 = degp_out[1].reshape(N_PAD, 1)
    di0 = degp_in[0].reshape(N_PAD, 1)
    di1 = degp_in[1].reshape(N_PAD, 1)

    y1 = _tc_mm1(x_p, do0, do1, W1)
    agg1 = _sc_agg(y1, src2d, dst2d)
    y2 = _tc_mid(agg1[0], agg1[1], di0, di1, do0, do1,
                 b1.reshape(1, F_H), W2)
    agg2 = _sc_agg(y2, src2d, dst2d)
    h2 = _tc_final(agg2[0], agg2[1], di0, di1, b2.reshape(1, F_H))
    return h2[:N]


# speedup vs baseline: 13.2923x; 1.0582x over previous
import functools

import jax
import jax.numpy as jnp
from jax import lax
from jax.experimental import pallas as pl
from jax.experimental.pallas import tpu as pltpu
from jax.experimental.pallas import tpu_sc as plsc

N = 10000
EDGES = 320000
F_IN = 128
F_H = 64

N_PAD = 10240
N_PER_TILE = N_PAD // 16
CHUNK = 128
N_TILES = 32
CPT = 80
E_PAD = N_TILES * CPT * CHUNK
ROW_BLK = 1280


def _mesh():
    return plsc.VectorSubcoreMesh(core_axis_name="c", subcore_axis_name="s")


def _sc_degrees(src2d, dst2d):
    def body(src_h, dst_h, dout_h, din_h, srcv, dstv, ones_v, zv, acc_o, acc_i):
        c = lax.axis_index("c")
        s = lax.axis_index("s")
        tid = s * 2 + c

        def set_ones(i, _):
            ones_v[pl.ds(i * 16, 16)] = jnp.ones((16,), jnp.float32)
            return 0

        lax.fori_loop(0, CHUNK // 16, set_ones, 0)

        def set_zero(i, _):
            zv[pl.ds(i * 16, 16)] = jnp.zeros((16,), jnp.float32)
            return 0

        lax.fori_loop(0, N_PER_TILE // 16, set_zero, 0)

        sl = pl.ds(s * N_PER_TILE, N_PER_TILE)
        pltpu.sync_copy(zv, acc_o.at[sl])
        pltpu.sync_copy(zv, acc_i.at[sl])
        plsc.subcore_barrier()

        pltpu.sync_copy(src_h.at[pl.ds(tid * CPT, CPT)], srcv)
        pltpu.sync_copy(dst_h.at[pl.ds(tid * CPT, CPT)], dstv)

        def step(j, _):
            pltpu.sync_copy(ones_v, acc_o.at[srcv.at[j]], add=True)
            pltpu.sync_copy(ones_v, acc_i.at[dstv.at[j]], add=True)
            return 0

        lax.fori_loop(0, CPT, step, 0)
        plsc.subcore_barrier()

        pltpu.sync_copy(acc_o.at[sl], dout_h.at[c, sl])
        pltpu.sync_copy(acc_i.at[sl], din_h.at[c, sl])

    return pl.kernel(
        body,
        out_type=[
            jax.ShapeDtypeStruct((2, N_PAD), jnp.float32),
            jax.ShapeDtypeStruct((2, N_PAD), jnp.float32),
        ],
        mesh=_mesh(),
        scratch_types=[
            pltpu.VMEM((CPT, CHUNK), jnp.int32),
            pltpu.VMEM((CPT, CHUNK), jnp.int32),
            pltpu.VMEM((CHUNK,), jnp.float32),
            pltpu.VMEM((N_PER_TILE,), jnp.float32),
            pltpu.VMEM_SHARED((N_PAD,), jnp.float32),
            pltpu.VMEM_SHARED((N_PAD,), jnp.float32),
        ],
    )(src2d, dst2d)


def _sc_agg(y, src2d, dst2d):
    def body(y_h, src_h, dst_h, out_h, srcv, dstv, rows0, rows1, zbuf, acc,
             sem0, sem1):
        c = lax.axis_index("c")
        s = lax.axis_index("s")
        tid = s * 2 + c

        def zb(i, _):
            zbuf[i // 4, pl.ds((i % 4) * 16, 16)] = jnp.zeros((16,), jnp.float32)
            return 0

        lax.fori_loop(0, CHUNK * 4, zb, 0)

        def zc(i, _):
            pltpu.sync_copy(zbuf, acc.at[pl.ds(s * N_PER_TILE + i * CHUNK, CHUNK)])
            return 0

        lax.fori_loop(0, N_PER_TILE // CHUNK, zc, 0)
        plsc.subcore_barrier()

        pltpu.sync_copy(src_h.at[pl.ds(tid * CPT, CPT)], srcv)
        pltpu.sync_copy(dst_h.at[pl.ds(tid * CPT, CPT)], dstv)

        pltpu.async_copy(y_h.at[srcv.at[0]], rows0, sem0)

        def step(g, _):
            j0 = g * 2
            pltpu.async_copy(y_h.at[srcv.at[j0 + 1]], rows1, sem1)
            pltpu.make_async_copy(y_h.at[srcv.at[j0]], rows0, sem0).wait()
            pltpu.sync_copy(rows0, acc.at[dstv.at[j0]], add=True)

            @pl.when(g < CPT // 2 - 1)
            def _issue_next():
                pltpu.async_copy(y_h.at[srcv.at[j0 + 2]], rows0, sem0)

            pltpu.make_async_copy(y_h.at[srcv.at[j0 + 1]], rows1, sem1).wait()
            pltpu.sync_copy(rows1, acc.at[dstv.at[j0 + 1]], add=True)
            return 0

        lax.fori_loop(0, CPT // 2, step, 0)
        plsc.subcore_barrier()

        sl = pl.ds(s * N_PER_TILE, N_PER_TILE)
        pltpu.sync_copy(acc.at[sl], out_h.at[c, sl])

    return pl.kernel(
        body,
        out_type=jax.ShapeDtypeStruct((2, N_PAD, F_H), jnp.float32),
        mesh=_mesh(),
        scratch_types=[
            pltpu.VMEM((CPT, CHUNK), jnp.int32),
            pltpu.VMEM((CPT, CHUNK), jnp.int32),
            pltpu.VMEM((CHUNK, F_H), jnp.float32),
            pltpu.VMEM((CHUNK, F_H), jnp.float32),
            pltpu.VMEM((CHUNK, F_H), jnp.float32),
            pltpu.VMEM_SHARED((N_PAD, F_H), jnp.float32),
            pltpu.SemaphoreType.DMA,
            pltpu.SemaphoreType.DMA,
        ],
        compiler_params=pltpu.CompilerParams(use_tc_tiling_on_sc=False),
    )(y, src2d, dst2d)


def _tc_mm1(x, dout, W1):
    def body(x_ref, d_ref, w_ref, o_ref):
        d = d_ref[0] + d_ref[1]
        ns = lax.rsqrt(jnp.maximum(d, 1.0))
        o_ref[...] = jnp.dot(x_ref[...] * ns, w_ref[...],
                             preferred_element_type=jnp.float32)

    return pl.pallas_call(
        body,
        grid=(N_PAD // ROW_BLK,),
        in_specs=[
            pl.BlockSpec((ROW_BLK, F_IN), lambda i: (i, 0)),
            pl.BlockSpec((2, ROW_BLK, 1), lambda i: (0, i, 0)),
            pl.BlockSpec((F_IN, F_H), lambda i: (0, 0)),
        ],
        out_specs=pl.BlockSpec((ROW_BLK, F_H), lambda i: (i, 0)),
        out_shape=jax.ShapeDtypeStruct((N_PAD, F_H), jnp.float32),
    )(x, dout, W1)


def _tc_mid(agg, din, dout, b1, W2):
    def body(a_ref, i_ref, o_ref2, br, w_ref, o_ref):
        nd = lax.rsqrt(jnp.maximum(i_ref[0] + i_ref[1], 1.0))
        h = jnp.maximum((a_ref[0] + a_ref[1]) * nd + br[...], 0.0)
        ns = lax.rsqrt(jnp.maximum(o_ref2[0] + o_ref2[1], 1.0))
        o_ref[...] = jnp.dot(h * ns, w_ref[...],
                             preferred_element_type=jnp.float32)

    return pl.pallas_call(
        body,
        grid=(N_PAD // ROW_BLK,),
        in_specs=[
            pl.BlockSpec((2, ROW_BLK, F_H), lambda i: (0, i, 0)),
            pl.BlockSpec((2, ROW_BLK, 1), lambda i: (0, i, 0)),
            pl.BlockSpec((2, ROW_BLK, 1), lambda i: (0, i, 0)),
            pl.BlockSpec((1, F_H), lambda i: (0, 0)),
            pl.BlockSpec((F_H, F_H), lambda i: (0, 0)),
        ],
        out_specs=pl.BlockSpec((ROW_BLK, F_H), lambda i: (i, 0)),
        out_shape=jax.ShapeDtypeStruct((N_PAD, F_H), jnp.float32),
    )(agg, din, dout, b1, W2)


def _tc_final(agg, din, b2):
    def body(a_ref, i_ref, br, o_ref):
        nd = lax.rsqrt(jnp.maximum(i_ref[0] + i_ref[1], 1.0))
        o_ref[...] = jnp.maximum((a_ref[0] + a_ref[1]) * nd + br[...], 0.0)

    blk = 1000
    return pl.pallas_call(
        body,
        grid=(N // blk,),
        in_specs=[
            pl.BlockSpec((2, blk, F_H), lambda i: (0, i, 0)),
            pl.BlockSpec((2, blk, 1), lambda i: (0, i, 0)),
            pl.BlockSpec((1, F_H), lambda i: (0, 0)),
        ],
        out_specs=pl.BlockSpec((blk, F_H), lambda i: (i, 0)),
        out_shape=jax.ShapeDtypeStruct((N, F_H), jnp.float32),
    )(agg, din, b2)


def kernel(inputs, edge_index, W1, b1, W2, b2):
    src = edge_index[0]
    dst = edge_index[1]
    pad = E_PAD - EDGES
    padidx = (N + (jnp.arange(pad, dtype=jnp.int32) % (N_PAD - N))).astype(jnp.int32)
    src2d = jnp.concatenate([src, padidx]).reshape(E_PAD // CHUNK, CHUNK)
    dst2d = jnp.concatenate([dst, padidx]).reshape(E_PAD // CHUNK, CHUNK)
    x_p = jnp.concatenate(
        [inputs, jnp.zeros((N_PAD - N, F_IN), jnp.float32)], axis=0)

    degp_out, degp_in = _sc_degrees(src2d, dst2d)
    dout = degp_out.reshape(2, N_PAD, 1)
    din = degp_in.reshape(2, N_PAD, 1)

    y1 = _tc_mm1(x_p, dout, W1)
    agg1 = _sc_agg(y1, src2d, dst2d)
    y2 = _tc_mid(agg1, din, dout, b1.reshape(1, F_H), W2)
    agg2 = _sc_agg(y2, src2d, dst2d)
    return _tc_final(agg2, din, b2.reshape(1, F_H))


# speedup vs baseline: 15.4995x; 1.1660x over previous
import functools

import jax
import jax.numpy as jnp
from jax import lax
from jax.experimental import pallas as pl
from jax.experimental.pallas import tpu as pltpu
from jax.experimental.pallas import tpu_sc as plsc

N = 10000
EDGES = 320000
F_IN = 128
F_H = 64

N_PAD = 10240
N_PER_TILE = N_PAD // 16
CHUNK = 128
N_TILES = 32
CPT = 80
E_PAD = N_TILES * CPT * CHUNK
ROW_BLK = 1280


def _mesh():
    return plsc.VectorSubcoreMesh(core_axis_name="c", subcore_axis_name="s")


def _sc_degrees(src2d, dst2d):
    def body(src_h, dst_h, dout_h, din_h, srcv, dstv, ones_v, zv, acc_o, acc_i,
             sem_a, sem_b):
        c = lax.axis_index("c")
        s = lax.axis_index("s")
        tid = s * 2 + c

        def set_ones(i, _):
            ones_v[pl.ds(i * 16, 16)] = jnp.ones((16,), jnp.float32)
            return 0

        lax.fori_loop(0, CHUNK // 16, set_ones, 0)

        def set_zero(i, _):
            zv[pl.ds(i * 16, 16)] = jnp.zeros((16,), jnp.float32)
            return 0

        lax.fori_loop(0, N_PER_TILE // 16, set_zero, 0)

        sl = pl.ds(s * N_PER_TILE, N_PER_TILE)
        pltpu.sync_copy(zv, acc_o.at[sl])
        pltpu.sync_copy(zv, acc_i.at[sl])
        plsc.subcore_barrier()

        pltpu.sync_copy(src_h.at[pl.ds(tid * CPT, CPT)], srcv)
        pltpu.sync_copy(dst_h.at[pl.ds(tid * CPT, CPT)], dstv)

        def step(j, _):
            pltpu.async_copy(ones_v, acc_o.at[srcv.at[j]], sem_a, add=True)
            pltpu.async_copy(ones_v, acc_i.at[dstv.at[j]], sem_b, add=True)

            @pl.when(j >= 16)
            def _drain_old():
                pltpu.make_async_copy(ones_v, acc_o.at[srcv.at[j - 16]], sem_a).wait()
                pltpu.make_async_copy(ones_v, acc_i.at[dstv.at[j - 16]], sem_b).wait()

            return 0

        lax.fori_loop(0, CPT, step, 0)

        def drain(j, _):
            pltpu.make_async_copy(ones_v, acc_o.at[srcv.at[CPT - 16 + j]], sem_a).wait()
            pltpu.make_async_copy(ones_v, acc_i.at[dstv.at[CPT - 16 + j]], sem_b).wait()
            return 0

        lax.fori_loop(0, 16, drain, 0)
        plsc.subcore_barrier()

        pltpu.sync_copy(acc_o.at[sl], dout_h.at[c, sl])
        pltpu.sync_copy(acc_i.at[sl], din_h.at[c, sl])

    return pl.kernel(
        body,
        out_type=[
            jax.ShapeDtypeStruct((2, N_PAD), jnp.float32),
            jax.ShapeDtypeStruct((2, N_PAD), jnp.float32),
        ],
        mesh=_mesh(),
        scratch_types=[
            pltpu.VMEM((CPT, CHUNK), jnp.int32),
            pltpu.VMEM((CPT, CHUNK), jnp.int32),
            pltpu.VMEM((CHUNK,), jnp.float32),
            pltpu.VMEM((N_PER_TILE,), jnp.float32),
            pltpu.VMEM_SHARED((N_PAD,), jnp.float32),
            pltpu.VMEM_SHARED((N_PAD,), jnp.float32),
            pltpu.SemaphoreType.DMA,
            pltpu.SemaphoreType.DMA,
        ],
    )(src2d, dst2d)


def _sc_agg(y, src2d, dst2d):
    NBUF = 8
    PF = 4

    def body(y_h, src_h, dst_h, out_h, srcv, dstv, rows, zbuf, acc, gsem, ssem):
        c = lax.axis_index("c")
        s = lax.axis_index("s")
        tid = s * 2 + c

        def zb(i, _):
            zbuf[i // 4, pl.ds((i % 4) * 16, 16)] = jnp.zeros((16,), jnp.float32)
            return 0

        lax.fori_loop(0, 16 * 4, zb, 0)

        def zc(i, _):
            pltpu.async_copy(
                zbuf, acc.at[pl.ds(s * N_PER_TILE + i * 16, 16)], gsem.at[0])
            return 0

        lax.fori_loop(0, N_PER_TILE // 16, zc, 0)

        def zw(i, _):
            pltpu.make_async_copy(
                zbuf, acc.at[pl.ds(s * N_PER_TILE + i * 16, 16)], gsem.at[0]).wait()
            return 0

        lax.fori_loop(0, N_PER_TILE // 16, zw, 0)
        plsc.subcore_barrier()

        pltpu.sync_copy(src_h.at[pl.ds(tid * CPT, CPT)], srcv)
        pltpu.sync_copy(dst_h.at[pl.ds(tid * CPT, CPT)], dstv)

        for b in range(PF):
            pltpu.async_copy(y_h.at[srcv.at[b]], rows.at[b], gsem.at[b])

        def step(g, _):
            for b in range(NBUF):
                j = g * NBUF + b
                bn = (b + PF) % NBUF

                @pl.when(jnp.logical_and(j >= PF, j < CPT - PF))
                def _wait_free():
                    pltpu.make_async_copy(
                        rows.at[bn], acc.at[dstv.at[j - PF]], ssem.at[bn]).wait()

                @pl.when(j < CPT - PF)
                def _prefetch():
                    pltpu.async_copy(
                        y_h.at[srcv.at[j + PF]], rows.at[bn], gsem.at[bn])

                pltpu.make_async_copy(
                    y_h.at[srcv.at[j]], rows.at[b], gsem.at[b]).wait()
                pltpu.async_copy(
                    rows.at[b], acc.at[dstv.at[j]], ssem.at[b], add=True)
            return 0

        lax.fori_loop(0, CPT // NBUF, step, 0)

        for b in range(NBUF):
            pltpu.make_async_copy(
                rows.at[b], acc.at[dstv.at[CPT - NBUF + b]], ssem.at[b]).wait()
        plsc.subcore_barrier()

        sl = pl.ds(s * N_PER_TILE, N_PER_TILE)
        pltpu.sync_copy(acc.at[sl], out_h.at[c, sl])

    return pl.kernel(
        body,
        out_type=jax.ShapeDtypeStruct((2, N_PAD, F_H), jnp.float32),
        mesh=_mesh(),
        scratch_types=[
            pltpu.VMEM((CPT, CHUNK), jnp.int32),
            pltpu.VMEM((CPT, CHUNK), jnp.int32),
            pltpu.VMEM((NBUF, CHUNK, F_H), jnp.float32),
            pltpu.VMEM((16, F_H), jnp.float32),
            pltpu.VMEM_SHARED((N_PAD, F_H), jnp.float32),
            pltpu.SemaphoreType.DMA((NBUF,)),
            pltpu.SemaphoreType.DMA((NBUF,)),
        ],
        compiler_params=pltpu.CompilerParams(use_tc_tiling_on_sc=False),
    )(y, src2d, dst2d)


def _tc_mm1(x, dout, W1):
    def body(x_ref, d_ref, w_ref, o_ref):
        d = d_ref[0] + d_ref[1]
        ns = lax.rsqrt(jnp.maximum(d, 1.0))
        o_ref[...] = jnp.dot(x_ref[...] * ns, w_ref[...],
                             preferred_element_type=jnp.float32)

    return pl.pallas_call(
        body,
        grid=(N_PAD // ROW_BLK,),
        in_specs=[
            pl.BlockSpec((ROW_BLK, F_IN), lambda i: (i, 0)),
            pl.BlockSpec((2, ROW_BLK, 1), lambda i: (0, i, 0)),
            pl.BlockSpec((F_IN, F_H), lambda i: (0, 0)),
        ],
        out_specs=pl.BlockSpec((ROW_BLK, F_H), lambda i: (i, 0)),
        out_shape=jax.ShapeDtypeStruct((N_PAD, F_H), jnp.float32),
    )(x, dout, W1)


def _tc_mid(agg, din, dout, b1, W2):
    def body(a_ref, i_ref, o_ref2, br, w_ref, o_ref):
        nd = lax.rsqrt(jnp.maximum(i_ref[0] + i_ref[1], 1.0))
        h = jnp.maximum((a_ref[0] + a_ref[1]) * nd + br[...], 0.0)
        ns = lax.rsqrt(jnp.maximum(o_ref2[0] + o_ref2[1], 1.0))
        o_ref[...] = jnp.dot(h * ns, w_ref[...],
                             preferred_element_type=jnp.float32)

    return pl.pallas_call(
        body,
        grid=(N_PAD // ROW_BLK,),
        in_specs=[
            pl.BlockSpec((2, ROW_BLK, F_H), lambda i: (0, i, 0)),
            pl.BlockSpec((2, ROW_BLK, 1), lambda i: (0, i, 0)),
            pl.BlockSpec((2, ROW_BLK, 1), lambda i: (0, i, 0)),
            pl.BlockSpec((1, F_H), lambda i: (0, 0)),
            pl.BlockSpec((F_H, F_H), lambda i: (0, 0)),
        ],
        out_specs=pl.BlockSpec((ROW_BLK, F_H), lambda i: (i, 0)),
        out_shape=jax.ShapeDtypeStruct((N_PAD, F_H), jnp.float32),
    )(agg, din, dout, b1, W2)


def _tc_final(agg, din, b2):
    def body(a_ref, i_ref, br, o_ref):
        nd = lax.rsqrt(jnp.maximum(i_ref[0] + i_ref[1], 1.0))
        o_ref[...] = jnp.maximum((a_ref[0] + a_ref[1]) * nd + br[...], 0.0)

    blk = 1000
    return pl.pallas_call(
        body,
        grid=(N // blk,),
        in_specs=[
            pl.BlockSpec((2, blk, F_H), lambda i: (0, i, 0)),
            pl.BlockSpec((2, blk, 1), lambda i: (0, i, 0)),
            pl.BlockSpec((1, F_H), lambda i: (0, 0)),
        ],
        out_specs=pl.BlockSpec((blk, F_H), lambda i: (i, 0)),
        out_shape=jax.ShapeDtypeStruct((N, F_H), jnp.float32),
    )(agg, din, b2)


def kernel(inputs, edge_index, W1, b1, W2, b2):
    src = edge_index[0]
    dst = edge_index[1]
    pad = E_PAD - EDGES
    padidx = (N + (jnp.arange(pad, dtype=jnp.int32) % (N_PAD - N))).astype(jnp.int32)
    src2d = jnp.concatenate([src, padidx]).reshape(E_PAD // CHUNK, CHUNK)
    dst2d = jnp.concatenate([dst, padidx]).reshape(E_PAD // CHUNK, CHUNK)
    x_p = jnp.concatenate(
        [inputs, jnp.zeros((N_PAD - N, F_IN), jnp.float32)], axis=0)

    degp_out, degp_in = _sc_degrees(src2d, dst2d)
    dout = degp_out.reshape(2, N_PAD, 1)
    din = degp_in.reshape(2, N_PAD, 1)

    y1 = _tc_mm1(x_p, dout, W1)
    agg1 = _sc_agg(y1, src2d, dst2d)
    y2 = _tc_mid(agg1, din, dout, b1.reshape(1, F_H), W2)
    agg2 = _sc_agg(y2, src2d, dst2d)
    return _tc_final(agg2, din, b2.reshape(1, F_H))


# speedup vs baseline: 15.5308x; 1.0020x over previous
import jax
import jax.numpy as jnp
from jax import lax
from jax.experimental import pallas as pl
from jax.experimental.pallas import tpu as pltpu
from jax.experimental.pallas import tpu_sc as plsc

N = 10000
EDGES = 320000
F_IN = 128
F_H = 64

N_PAD = 10240
N_PER_TILE = N_PAD // 16
CHUNK = 128
N_TILES = 32
NCHUNKS = EDGES // CHUNK
CPT = 80
LAST_RING = 16
LAST_TAIL = NCHUNKS - 31 * CPT - LAST_RING
ROW_BLK = 1000

NBUF = 8
PF = 4


def _mesh():
    return plsc.VectorSubcoreMesh(core_axis_name="c", subcore_axis_name="s")


def _stage_indices(src_h, dst_h, srcv, dstv, tid):
    @pl.when(tid < 31)
    def _full():
        pltpu.sync_copy(src_h.at[pl.ds(tid * CPT, CPT)], srcv.at[pl.ds(0, CPT)])
        pltpu.sync_copy(dst_h.at[pl.ds(tid * CPT, CPT)], dstv.at[pl.ds(0, CPT)])

    @pl.when(tid == 31)
    def _last():
        nlast = LAST_RING + LAST_TAIL
        pltpu.sync_copy(src_h.at[pl.ds(31 * CPT, nlast)], srcv.at[pl.ds(0, nlast)])
        pltpu.sync_copy(dst_h.at[pl.ds(31 * CPT, nlast)], dstv.at[pl.ds(0, nlast)])


def _sc_degrees(src2d, dst2d):
    def body(src_h, dst_h, dout_h, din_h, srcv, dstv, ones_v, zv, acc_o, acc_i,
             sem_a, sem_b):
        c = lax.axis_index("c")
        s = lax.axis_index("s")
        tid = s * 2 + c
        nt = jnp.where(tid < 31, CPT, LAST_RING + LAST_TAIL)

        def set_ones(i, _):
            ones_v[pl.ds(i * 16, 16)] = jnp.ones((16,), jnp.float32)
            return 0

        lax.fori_loop(0, CHUNK // 16, set_ones, 0)

        def set_zero(i, _):
            zv[pl.ds(i * 16, 16)] = jnp.zeros((16,), jnp.float32)
            return 0

        lax.fori_loop(0, N_PER_TILE // 16, set_zero, 0)

        sl = pl.ds(s * N_PER_TILE, N_PER_TILE)
        pltpu.sync_copy(zv, acc_o.at[sl])
        pltpu.sync_copy(zv, acc_i.at[sl])
        plsc.subcore_barrier()

        _stage_indices(src_h, dst_h, srcv, dstv, tid)

        def step(j, _):
            pltpu.async_copy(ones_v, acc_o.at[srcv.at[j]], sem_a, add=True)
            pltpu.async_copy(ones_v, acc_i.at[dstv.at[j]], sem_b, add=True)

            @pl.when(j >= 16)
            def _drain_old():
                pltpu.make_async_copy(ones_v, acc_o.at[srcv.at[j - 16]], sem_a).wait()
                pltpu.make_async_copy(ones_v, acc_i.at[dstv.at[j - 16]], sem_b).wait()

            return 0

        lax.fori_loop(0, nt, step, 0)

        def drain(i, _):
            pltpu.make_async_copy(ones_v, acc_o.at[srcv.at[nt - 16 + i]], sem_a).wait()
            pltpu.make_async_copy(ones_v, acc_i.at[dstv.at[nt - 16 + i]], sem_b).wait()
            return 0

        lax.fori_loop(0, 16, drain, 0)
        plsc.subcore_barrier()

        pltpu.sync_copy(acc_o.at[sl], dout_h.at[c, sl])
        pltpu.sync_copy(acc_i.at[sl], din_h.at[c, sl])

    return pl.kernel(
        body,
        out_type=[
            jax.ShapeDtypeStruct((2, N_PAD), jnp.float32),
            jax.ShapeDtypeStruct((2, N_PAD), jnp.float32),
        ],
        mesh=_mesh(),
        scratch_types=[
            pltpu.VMEM((CPT, CHUNK), jnp.int32),
            pltpu.VMEM((CPT, CHUNK), jnp.int32),
            pltpu.VMEM((CHUNK,), jnp.float32),
            pltpu.VMEM((N_PER_TILE,), jnp.float32),
            pltpu.VMEM_SHARED((N_PAD,), jnp.float32),
            pltpu.VMEM_SHARED((N_PAD,), jnp.float32),
            pltpu.SemaphoreType.DMA,
            pltpu.SemaphoreType.DMA,
        ],
    )(src2d, dst2d)


def _sc_agg(y, src2d, dst2d):
    def body(y_h, src_h, dst_h, out_h, srcv, dstv, rows, zbuf, acc, gsem, ssem):
        c = lax.axis_index("c")
        s = lax.axis_index("s")
        tid = s * 2 + c

        def zb(i, _):
            zbuf[i // 4, pl.ds((i % 4) * 16, 16)] = jnp.zeros((16,), jnp.float32)
            return 0

        lax.fori_loop(0, 16 * 4, zb, 0)

        def zc(i, _):
            pltpu.async_copy(
                zbuf, acc.at[pl.ds(s * N_PER_TILE + i * 16, 16)], gsem.at[0])
            return 0

        lax.fori_loop(0, N_PER_TILE // 16, zc, 0)

        def zw(i, _):
            pltpu.make_async_copy(
                zbuf, acc.at[pl.ds(s * N_PER_TILE + i * 16, 16)], gsem.at[0]).wait()
            return 0

        lax.fori_loop(0, N_PER_TILE // 16, zw, 0)
        plsc.subcore_barrier()

        _stage_indices(src_h, dst_h, srcv, dstv, tid)
        nring = jnp.where(tid < 31, CPT, LAST_RING)

        for b in range(PF):
            pltpu.async_copy(y_h.at[srcv.at[b]], rows.at[b], gsem.at[b])

        def step(g, _):
            for b in range(NBUF):
                j = g * NBUF + b
                bn = (b + PF) % NBUF

                @pl.when(jnp.logical_and(j >= PF, j < nring - PF))
                def _wait_free():
                    pltpu.make_async_copy(
                        rows.at[bn], acc.at[dstv.at[j - PF]], ssem.at[bn]).wait()

                @pl.when(j < nring - PF)
                def _prefetch():
                    pltpu.async_copy(
                        y_h.at[srcv.at[j + PF]], rows.at[bn], gsem.at[bn])

                pltpu.make_async_copy(
                    y_h.at[srcv.at[j]], rows.at[b], gsem.at[b]).wait()
                pltpu.async_copy(
                    rows.at[b], acc.at[dstv.at[j]], ssem.at[b], add=True)
            return 0

        lax.fori_loop(0, nring // NBUF, step, 0)

        for i in range(NBUF):
            k = nring - NBUF + i
            pltpu.make_async_copy(
                rows.at[i], acc.at[dstv.at[k]], ssem.at[i]).wait()

        @pl.when(tid == 31)
        def _tail():
            for t in range(LAST_TAIL):
                pltpu.sync_copy(y_h.at[srcv.at[LAST_RING + t]], rows.at[0])
                pltpu.sync_copy(rows.at[0], acc.at[dstv.at[LAST_RING + t]],
                                add=True)

        plsc.subcore_barrier()

        sl = pl.ds(s * N_PER_TILE, N_PER_TILE)
        pltpu.sync_copy(acc.at[sl], out_h.at[c, sl])

    return pl.kernel(
        body,
        out_type=jax.ShapeDtypeStruct((2, N_PAD, F_H), jnp.float32),
        mesh=_mesh(),
        scratch_types=[
            pltpu.VMEM((CPT, CHUNK), jnp.int32),
            pltpu.VMEM((CPT, CHUNK), jnp.int32),
            pltpu.VMEM((NBUF, CHUNK, F_H), jnp.float32),
            pltpu.VMEM((16, F_H), jnp.float32),
            pltpu.VMEM_SHARED((N_PAD, F_H), jnp.float32),
            pltpu.SemaphoreType.DMA((NBUF,)),
            pltpu.SemaphoreType.DMA((NBUF,)),
        ],
        compiler_params=pltpu.CompilerParams(use_tc_tiling_on_sc=False),
    )(y, src2d, dst2d)


def _tc_matmul(x, W1):
    def body(x_ref, w_ref, o_ref):
        o_ref[...] = jnp.dot(x_ref[...], w_ref[...],
                             preferred_element_type=jnp.float32)

    return pl.pallas_call(
        body,
        grid=(N // ROW_BLK,),
        in_specs=[
            pl.BlockSpec((ROW_BLK, F_IN), lambda i: (i, 0)),
            pl.BlockSpec((F_IN, F_H), lambda i: (0, 0)),
        ],
        out_specs=pl.BlockSpec((ROW_BLK, F_H), lambda i: (i, 0)),
        out_shape=jax.ShapeDtypeStruct((N, F_H), jnp.float32),
    )(x, W1)


def _tc_scale(z, dout):
    def body(z_ref, d_ref, o_ref):
        ns = lax.rsqrt(jnp.maximum(d_ref[0] + d_ref[1], 1.0))
        o_ref[...] = z_ref[...] * ns

    return pl.pallas_call(
        body,
        grid=(N // ROW_BLK,),
        in_specs=[
            pl.BlockSpec((ROW_BLK, F_H), lambda i: (i, 0)),
            pl.BlockSpec((2, ROW_BLK, 1), lambda i: (0, i, 0)),
        ],
        out_specs=pl.BlockSpec((ROW_BLK, F_H), lambda i: (i, 0)),
        out_shape=jax.ShapeDtypeStruct((N, F_H), jnp.float32),
    )(z, dout)


def _tc_mid(agg, din, dout, b1, W2):
    def body(a_ref, i_ref, o_ref2, br, w_ref, o_ref):
        nd = lax.rsqrt(jnp.maximum(i_ref[0] + i_ref[1], 1.0))
        h = jnp.maximum((a_ref[0] + a_ref[1]) * nd + br[...], 0.0)
        ns = lax.rsqrt(jnp.maximum(o_ref2[0] + o_ref2[1], 1.0))
        o_ref[...] = jnp.dot(h * ns, w_ref[...],
                             preferred_element_type=jnp.float32)

    return pl.pallas_call(
        body,
        grid=(N // ROW_BLK,),
        in_specs=[
            pl.BlockSpec((2, ROW_BLK, F_H), lambda i: (0, i, 0)),
            pl.BlockSpec((2, ROW_BLK, 1), lambda i: (0, i, 0)),
            pl.BlockSpec((2, ROW_BLK, 1), lambda i: (0, i, 0)),
            pl.BlockSpec((1, F_H), lambda i: (0, 0)),
            pl.BlockSpec((F_H, F_H), lambda i: (0, 0)),
        ],
        out_specs=pl.BlockSpec((ROW_BLK, F_H), lambda i: (i, 0)),
        out_shape=jax.ShapeDtypeStruct((N, F_H), jnp.float32),
    )(agg, din, dout, b1, W2)


def _tc_final(agg, din, b2):
    def body(a_ref, i_ref, br, o_ref):
        nd = lax.rsqrt(jnp.maximum(i_ref[0] + i_ref[1], 1.0))
        o_ref[...] = jnp.maximum((a_ref[0] + a_ref[1]) * nd + br[...], 0.0)

    return pl.pallas_call(
        body,
        grid=(N // ROW_BLK,),
        in_specs=[
            pl.BlockSpec((2, ROW_BLK, F_H), lambda i: (0, i, 0)),
            pl.BlockSpec((2, ROW_BLK, 1), lambda i: (0, i, 0)),
            pl.BlockSpec((1, F_H), lambda i: (0, 0)),
        ],
        out_specs=pl.BlockSpec((ROW_BLK, F_H), lambda i: (i, 0)),
        out_shape=jax.ShapeDtypeStruct((N, F_H), jnp.float32),
    )(agg, din, b2)


def kernel(inputs, edge_index, W1, b1, W2, b2):
    src2d = edge_index[0].reshape(NCHUNKS, CHUNK)
    dst2d = edge_index[1].reshape(NCHUNKS, CHUNK)

    degp_out, degp_in = _sc_degrees(src2d, dst2d)
    dout = degp_out.reshape(2, N_PAD, 1)
    din = degp_in.reshape(2, N_PAD, 1)

    z1 = _tc_matmul(inputs, W1)
    y1 = _tc_scale(z1, dout)
    agg1 = _sc_agg(y1, src2d, dst2d)
    y2 = _tc_mid(agg1, din, dout, b1.reshape(1, F_H), W2)
    agg2 = _sc_agg(y2, src2d, dst2d)
    return _tc_final(agg2, din, b2.reshape(1, F_H))
